# Initial kernel scaffold; baseline (speedup 1.0000x reference)
import jax
import jax.numpy as jnp
from jax.experimental import pallas as pl


def kernel(fc1_w, fc1_b, d1_w, d1_b, d2_w, d2_b, d3_w, d3_b, d4_w, d4_b, z):
    raise NotImplementedError("write your pallas kernel here")


# speedup vs baseline: 26.9140x; 26.9140x over previous
import jax
import jax.numpy as jnp
from jax.experimental import pallas as pl
from jax.experimental.pallas import tpu as pltpu

_VMEM_LIMIT = 56 * 1024 * 1024
_CORES = 2
_F32 = jnp.float32
_BF16 = jnp.bfloat16


def _g_mats(w_t, W):
    Cin, Cout, KH, KW = w_t.shape
    Wo = 2 * (W - 1) + KW
    w_i = jnp.arange(W)[:, None, None]
    wo_i = jnp.arange(Wo)[None, :, None]
    kw_i = jnp.arange(KW)[None, None, :]
    sel = (wo_i == 2 * w_i + kw_i).astype(w_t.dtype)
    g = jnp.einsum('wok,cdhk->hwcod', sel, w_t)
    return g.reshape(KH, W * Cin, Wo * Cout).astype(_BF16)


def _r_mats(Bh, H, KH):
    Ho = 2 * (H - 1) + KH
    kh = jnp.arange(KH)[:, None, None]
    ro = jnp.arange(Bh * Ho)[None, :, None]
    ri = jnp.arange(Bh * H)[None, None, :]
    m = (ro // Ho == ri // H) & (ro % Ho == 2 * (ri % H) + kh)
    return m.astype(_F32)


def _q_mats(B, Bh, KH):
    core = jnp.arange(_CORES)[:, None, None, None]
    kh = jnp.arange(KH)[None, :, None, None]
    r = jnp.arange(Bh * KH)[None, None, :, None]
    b = jnp.arange(B)[None, None, None, :]
    m = (r % KH == kh) & (b == Bh * core + r // KH)
    return m.astype(_F32)


def _layer(x, g_ref, r_ref, brow_ref, KH, act):
    acc = None
    for kh in range(KH):
        y = jnp.dot(x, g_ref[kh], preferred_element_type=_F32)
        t = jnp.dot(r_ref[0, kh], y, preferred_element_type=_F32)
        acc = t if acc is None else acc + t
    return act(acc + brow_ref[...])


def _decoder_kernel(z_ref, w1_ref, b1_ref, w2_ref, brow1_ref, q1_ref,
                    g2_ref, r2_ref, brow2_ref,
                    g3_ref, r3_ref, brow3_ref,
                    g4_ref, r4_ref, brow4_ref, o_ref):
    relu = lambda v: jnp.maximum(v, 0.0)
    h = jnp.dot(z_ref[...], w1_ref[...], preferred_element_type=_F32)
    h = relu(h + b1_ref[...])
    x = None
    for kh in range(5):
        t = relu(jnp.dot(h, w2_ref[kh], preferred_element_type=_F32)
                 + brow1_ref[...])
        t = jnp.dot(q1_ref[0, kh], t, preferred_element_type=_F32)
        x = t if x is None else x + t
    x = _layer(x, g2_ref, r2_ref, brow2_ref, 5, relu)
    x = _layer(x, g3_ref, r3_ref, brow3_ref, 6, relu)
    x = _layer(x, g4_ref, r4_ref, brow4_ref, 6, jax.nn.sigmoid)
    o_ref[...] = x


def kernel(fc1_w, fc1_b, d1_w, d1_b, d2_w, d2_b, d3_w, d3_b, d4_w, d4_b, z):
    B, L = z.shape
    Lh = fc1_w.shape[1]
    Cimg = d4_w.shape[1]
    Bh = B // _CORES

    w1 = fc1_w.astype(_F32)
    b1 = fc1_b.reshape(1, Lh).astype(_F32)
    w2 = d1_w.transpose(2, 0, 3, 1).reshape(5, d1_w.shape[0], -1).astype(_BF16)
    brow1 = jnp.tile(d1_b, 5).reshape(1, -1).astype(_F32)
    q1 = _q_mats(B, Bh, 5)

    def brow(b, Wo):
        return jnp.tile(b, Wo).reshape(1, -1).astype(_F32)

    g2, r2, brow2 = _g_mats(d2_w, 5), _r_mats(Bh, 5, 5), brow(d2_b, 13)
    g3, r3, brow3 = _g_mats(d3_w, 13), _r_mats(Bh, 13, 6), brow(d3_b, 30)
    g4, r4, brow4 = _g_mats(d4_w, 30), _r_mats(Bh, 30, 6), brow(d4_b, 64)
    r2 = jnp.broadcast_to(r2[None], (_CORES,) + r2.shape)
    r3 = jnp.broadcast_to(r3[None], (_CORES,) + r3.shape)
    r4 = jnp.broadcast_to(r4[None], (_CORES,) + r4.shape)

    full = lambda arr: pl.BlockSpec(arr.shape, lambda i: (0,) * arr.ndim)
    percore = lambda arr: pl.BlockSpec((1,) + arr.shape[1:],
                                       lambda i: (i,) + (0,) * (arr.ndim - 1))

    out = pl.pallas_call(
        _decoder_kernel,
        out_shape=jax.ShapeDtypeStruct((B * 64, 64 * Cimg), _F32),
        grid=(_CORES,),
        in_specs=[
            pl.BlockSpec((B, L), lambda i: (0, 0)),
            full(w1), full(b1), full(w2), full(brow1),
            percore(q1),
            full(g2), percore(r2), full(brow2),
            full(g3), percore(r3), full(brow3),
            full(g4), percore(r4), full(brow4),
        ],
        out_specs=pl.BlockSpec((Bh * 64, 64 * Cimg), lambda i: (i, 0)),
        compiler_params=pltpu.CompilerParams(
            dimension_semantics=("parallel",),
            vmem_limit_bytes=_VMEM_LIMIT,
        ),
    )(z.astype(_F32), w1, b1, w2, brow1, q1,
      g2, r2, brow2, g3, r3, brow3, g4, r4, brow4)

    return out.reshape(B, 64, 64, Cimg).transpose(0, 3, 1, 2)


# speedup vs baseline: 26.9712x; 1.0021x over previous
import jax
import jax.numpy as jnp
from jax.experimental import pallas as pl
from jax.experimental.pallas import tpu as pltpu

_VMEM_LIMIT = 56 * 1024 * 1024
_CORES = 2
_F32 = jnp.float32
_BF16 = jnp.bfloat16


def _g_mats(w_t, W):
    Cin, Cout, KH, KW = w_t.shape
    Wo = 2 * (W - 1) + KW
    w_i = jnp.arange(W)[:, None, None]
    wo_i = jnp.arange(Wo)[None, :, None]
    kw_i = jnp.arange(KW)[None, None, :]
    sel = (wo_i == 2 * w_i + kw_i).astype(_BF16)
    g = jnp.einsum('wok,cdhk->hwcod', sel, w_t.astype(_BF16),
                   preferred_element_type=_BF16)
    return g.reshape(KH, W * Cin, Wo * Cout)


def _r_mats(Bh, H, KH):
    Ho = 2 * (H - 1) + KH
    kh = jnp.arange(KH)[:, None, None]
    ro = jnp.arange(Bh * Ho)[None, :, None]
    ri = jnp.arange(Bh * H)[None, None, :]
    m = (ro // Ho == ri // H) & (ro % Ho == 2 * (ri % H) + kh)
    return m.astype(_F32)


def _q_mats(B, Bh, KH):
    core = jnp.arange(_CORES)[:, None, None, None]
    kh = jnp.arange(KH)[None, :, None, None]
    r = jnp.arange(Bh * KH)[None, None, :, None]
    b = jnp.arange(B)[None, None, None, :]
    m = (r % KH == kh) & (b == Bh * core + r // KH)
    return m.astype(_F32)


def _layer(x, g_ref, r_ref, brow_ref, KH, act):
    acc = None
    for kh in range(KH):
        y = jnp.dot(x, g_ref[kh], preferred_element_type=_F32)
        t = jnp.dot(r_ref[kh], y, preferred_element_type=_F32)
        acc = t if acc is None else acc + t
    return act(acc + brow_ref[...])


def _decoder_kernel(z_ref, w1_ref, b1_ref, w2_ref, brow1_ref, q1_ref,
                    g2_ref, r2_ref, brow2_ref,
                    g3_ref, r3_ref, brow3_ref,
                    g4_ref, r4_ref, brow4_ref, o_ref):
    relu = lambda v: jnp.maximum(v, 0.0)
    h = jnp.dot(z_ref[...], w1_ref[...], preferred_element_type=_F32)
    h = relu(h + b1_ref[...])
    x = None
    for kh in range(5):
        t = relu(jnp.dot(h, w2_ref[kh], preferred_element_type=_F32)
                 + brow1_ref[...])
        t = jnp.dot(q1_ref[0, kh], t, preferred_element_type=_F32)
        x = t if x is None else x + t
    x = _layer(x, g2_ref, r2_ref, brow2_ref, 5, relu)
    x = _layer(x, g3_ref, r3_ref, brow3_ref, 6, relu)
    x = _layer(x, g4_ref, r4_ref, brow4_ref, 6, jax.nn.sigmoid)
    o_ref[...] = x


def kernel(fc1_w, fc1_b, d1_w, d1_b, d2_w, d2_b, d3_w, d3_b, d4_w, d4_b, z):
    B, L = z.shape
    Lh = fc1_w.shape[1]
    Cimg = d4_w.shape[1]
    Bh = B // _CORES

    w1 = fc1_w.astype(_F32)
    b1 = fc1_b.reshape(1, Lh).astype(_F32)
    w2 = d1_w.astype(_BF16).transpose(2, 0, 3, 1).reshape(5, d1_w.shape[0], -1)
    brow1 = jnp.tile(d1_b, 5).reshape(1, -1).astype(_F32)
    q1 = _q_mats(B, Bh, 5)

    def brow(b, Wo):
        return jnp.tile(b, Wo).reshape(1, -1).astype(_F32)

    g2, r2, brow2 = _g_mats(d2_w, 5), _r_mats(Bh, 5, 5), brow(d2_b, 13)
    g3, r3, brow3 = _g_mats(d3_w, 13), _r_mats(Bh, 13, 6), brow(d3_b, 30)
    g4, r4, brow4 = _g_mats(d4_w, 30), _r_mats(Bh, 30, 6), brow(d4_b, 64)

    full = lambda arr: pl.BlockSpec(arr.shape, lambda i: (0,) * arr.ndim)
    percore = lambda arr: pl.BlockSpec((1,) + arr.shape[1:],
                                       lambda i: (i,) + (0,) * (arr.ndim - 1))

    out = pl.pallas_call(
        _decoder_kernel,
        out_shape=jax.ShapeDtypeStruct((B * 64, 64 * Cimg), _F32),
        grid=(_CORES,),
        in_specs=[
            pl.BlockSpec((B, L), lambda i: (0, 0)),
            full(w1), full(b1), full(w2), full(brow1),
            percore(q1),
            full(g2), full(r2), full(brow2),
            full(g3), full(r3), full(brow3),
            full(g4), full(r4), full(brow4),
        ],
        out_specs=pl.BlockSpec((Bh * 64, 64 * Cimg), lambda i: (i, 0)),
        compiler_params=pltpu.CompilerParams(
            dimension_semantics=("parallel",),
            vmem_limit_bytes=_VMEM_LIMIT,
        ),
    )(z.astype(_F32), w1, b1, w2, brow1, q1,
      g2, r2, brow2, g3, r3, brow3, g4, r4, brow4)

    return out.reshape(B, 64, 64, Cimg).transpose(0, 3, 1, 2)
